# baseline (speedup 1.0000x reference)
import functools

import jax
import jax.numpy as jnp
from jax import lax
from jax.experimental import pallas as pl
from jax.experimental.pallas import tpu as pltpu
from jax.experimental.pallas import tpu_sc as plsc

NUM_FIELDS = 26
VOCAB = 100000
EMB_DIM = 16
DENSE_DIM = 13
BATCH = 4096
H1, H2 = 64, 32
BN_EPS = 1e-5

FM_COLS = NUM_FIELDS * EMB_DIM
X_COLS = 512

_NC, _NS = 2, 16
_NW = _NC * _NS
_BF = BATCH * NUM_FIELDS
_ROWS_PER_W = _BF // _NW
_B_PER_W = BATCH // _NW
_CHUNK_B = 16
_CHUNK_ROWS = _CHUNK_B * NUM_FIELDS
_N_CHUNKS = _B_PER_W // _CHUNK_B


def _sc_fm_gather(fm128, r_idx, c_idx, jbc):
    mesh = plsc.VectorSubcoreMesh(core_axis_name="c", subcore_axis_name="s")

    @functools.partial(
        pl.kernel,
        mesh=mesh,
        compiler_params=pltpu.CompilerParams(needs_layout_passes=False),
        out_type=jax.ShapeDtypeStruct((BATCH, X_COLS), jnp.float32),
        scratch_types=[
            pltpu.VMEM((_ROWS_PER_W,), jnp.int32),
            pltpu.VMEM((_ROWS_PER_W,), jnp.int32),
            pltpu.VMEM((3, _CHUNK_ROWS), jnp.int32),
            pltpu.VMEM((_CHUNK_ROWS, 128), jnp.float32),
            pltpu.VMEM((_CHUNK_B, X_COLS), jnp.float32),
            pltpu.SemaphoreType.DMA,
        ],
    )
    def k(tab, r_hbm, c_hbm, jbc_hbm, out, rv, cv, pv, buf, stage, sem):
        wid = lax.axis_index("s") * _NC + lax.axis_index("c")
        base = wid * _ROWS_PER_W
        b0 = wid * _B_PER_W
        pltpu.sync_copy(r_hbm.at[pl.ds(base, _ROWS_PER_W)], rv)
        pltpu.sync_copy(c_hbm.at[pl.ds(base, _ROWS_PER_W)], cv)
        pltpu.sync_copy(jbc_hbm, pv)
        zero = jnp.zeros((16,), jnp.float32)
        for b in range(_CHUNK_B):
            for col in range(FM_COLS, X_COLS, 16):
                stage[b, pl.ds(col, 16)] = zero

        def chunk_body(ch, _):
            off = ch * _CHUNK_ROWS
            cp = pltpu.async_copy(
                tab.at[rv.at[pl.ds(off, _CHUNK_ROWS)]], buf, sem)
            cp.wait()
            for g in range(_CHUNK_ROWS // 16):
                j_vec = pv[0, pl.ds(g * 16, 16)]
                b_vec = pv[1, pl.ds(g * 16, 16)]
                col_vec = pv[2, pl.ds(g * 16, 16)]
                c_vec = cv[pl.ds(off + g * 16, 16)]
                for d in range(EMB_DIM):
                    vals = plsc.load_gather(buf, [j_vec, c_vec + d])
                    plsc.store_scatter(stage, [b_vec, col_vec + d], vals)
            pltpu.sync_copy(stage, out.at[pl.ds(b0 + ch * _CHUNK_B,
                                                _CHUNK_B)])
            return _

        lax.fori_loop(0, _N_CHUNKS, chunk_body, None)

    return k(fm128, r_idx, c_idx, jbc)


def _sc_lin_gather(lin_flat, idx_flat):
    mesh = plsc.VectorSubcoreMesh(core_axis_name="c", subcore_axis_name="s")

    @functools.partial(
        pl.kernel,
        mesh=mesh,
        compiler_params=pltpu.CompilerParams(use_tc_tiling_on_sc=False),
        out_type=jax.ShapeDtypeStruct((_BF,), jnp.float32),
        scratch_types=[
            pltpu.VMEM((_ROWS_PER_W,), jnp.int32),
            pltpu.VMEM((_ROWS_PER_W,), jnp.float32),
            pltpu.SemaphoreType.DMA,
        ],
    )
    def k(lin_hbm, idx_hbm, lin_out, idx_v, lin_v, sem):
        wid = lax.axis_index("s") * _NC + lax.axis_index("c")
        base = wid * _ROWS_PER_W
        pltpu.sync_copy(idx_hbm.at[pl.ds(base, _ROWS_PER_W)], idx_v)
        pltpu.async_copy(lin_hbm.at[idx_v], lin_v, sem).wait()
        pltpu.sync_copy(lin_v, lin_out.at[pl.ds(base, _ROWS_PER_W)])

    return k(lin_flat, idx_flat)


def _tc_body(x_ref, dense_ref, lin_ref, sel_ref, wd_ref, w1a_ref, w1b_ref,
             b1_ref, g1_ref, bt1_ref, w2_ref, b2_ref, g2_ref, bt2_ref,
             wout_ref, cbias_ref, out_ref):
    f32 = jnp.float32
    x = x_ref[...]
    d = dense_ref[...]
    sel = sel_ref[...]
    dn = (((1,), (1,)), ((), ()))

    sv = jnp.dot(x, sel, preferred_element_type=f32)
    sq = jnp.dot(x * x, sel, preferred_element_type=f32)
    fm_logit = 0.5 * jnp.sum(sv * sv - sq, axis=1, keepdims=True)

    lin_logit = jnp.sum(lin_ref[...], axis=1, keepdims=True)
    lin_logit = lin_logit + lax.dot_general(d, wd_ref[...], dn,
                                            preferred_element_type=f32)

    inv = lax.rsqrt(jnp.float32(1.0 + BN_EPS))
    z = lax.dot_general(x, w1a_ref[...], dn, preferred_element_type=f32)
    z = z + lax.dot_general(d, w1b_ref[...], dn, preferred_element_type=f32)
    h = jnp.maximum((z + b1_ref[...]) * (g1_ref[...] * inv) + bt1_ref[...],
                    0.0)
    z2 = lax.dot_general(h, w2_ref[...], dn, preferred_element_type=f32)
    h2 = jnp.maximum((z2 + b2_ref[...]) * (g2_ref[...] * inv) + bt2_ref[...],
                     0.0)
    dnn_logit = lax.dot_general(h2, wout_ref[...], dn,
                                preferred_element_type=f32)

    total = lin_logit + fm_logit + dnn_logit + cbias_ref[...]
    out_ref[...] = jax.nn.sigmoid(total)


def _tc_dense(x, dense_inputs, lin_vals, sel, wd, w1a, w1b, b1, g1, bt1, w2,
              b2, g2, bt2, wout, cbias):
    bm = 1024
    grid = (BATCH // bm,)
    full = lambda shape: pl.BlockSpec(shape, lambda i: (0,) * len(shape))
    row = lambda cols: pl.BlockSpec((bm, cols), lambda i: (i, 0))
    return pl.pallas_call(
        _tc_body,
        grid=grid,
        in_specs=[
            row(X_COLS),
            row(DENSE_DIM),
            row(NUM_FIELDS),
            full((X_COLS, EMB_DIM)),
            full((1, DENSE_DIM)),
            full((H1, X_COLS)),
            full((H1, DENSE_DIM)),
            full((1, H1)), full((1, H1)), full((1, H1)),
            full((H2, H1)),
            full((1, H2)), full((1, H2)), full((1, H2)),
            full((1, H2)),
            full((1, 1)),
        ],
        out_specs=row(1),
        out_shape=jax.ShapeDtypeStruct((BATCH, 1), jnp.float32),
    )(x, dense_inputs, lin_vals, sel, wd, w1a, w1b, b1, g1, bt1, w2, b2, g2,
      bt2, wout, cbias)


def kernel(sparse_inputs, dense_inputs, fm_tables, lin_tables, W_dense,
           b_dense, bias, W1, b1, g1, bt1, W2, b2, g2, bt2, Wout, bout):
    i32 = jnp.int32
    q = (sparse_inputs.astype(i32)
         + (jnp.arange(NUM_FIELDS, dtype=i32) * VOCAB)[None, :]).reshape(-1)
    r_idx = q >> 3
    c_idx = (q & 7) << 4
    fm128 = fm_tables.reshape(NUM_FIELDS * VOCAB * EMB_DIM // 128, 128)
    lin_flat = lin_tables.reshape(NUM_FIELDS * VOCAB)

    j_pat = jnp.arange(_CHUNK_ROWS, dtype=i32)
    b_pat = j_pat // NUM_FIELDS
    col_pat = (j_pat - b_pat * NUM_FIELDS) * EMB_DIM
    jbc = jnp.stack([j_pat, b_pat, col_pat], axis=0)

    x = _sc_fm_gather(fm128, r_idx, c_idx, jbc)
    lin_vals = _sc_lin_gather(lin_flat, q)
    lin_mat = lin_vals.reshape(BATCH, NUM_FIELDS)

    sel = jnp.concatenate(
        [jnp.tile(jnp.eye(EMB_DIM, dtype=jnp.float32), (NUM_FIELDS, 1)),
         jnp.zeros((X_COLS - FM_COLS, EMB_DIM), jnp.float32)], axis=0)
    w1a = jnp.concatenate(
        [W1[:, :FM_COLS],
         jnp.zeros((H1, X_COLS - FM_COLS), jnp.float32)], axis=1)
    w1b = W1[:, FM_COLS:]
    cbias = (bias + b_dense + bout).reshape(1, 1)
    out = _tc_dense(
        x, dense_inputs, lin_mat, sel, W_dense, w1a, w1b,
        b1.reshape(1, H1), g1.reshape(1, H1), bt1.reshape(1, H1),
        W2, b2.reshape(1, H2), g2.reshape(1, H2), bt2.reshape(1, H2),
        Wout, cbias)
    return out.reshape(BATCH)
